# Initial kernel scaffold; baseline (speedup 1.0000x reference)
import jax
import jax.numpy as jnp
from jax.experimental import pallas as pl


def kernel(x, edge_index, W_self0, W_neigh0, b0, W_self1, W_neigh1, b1):
    raise NotImplementedError("write your pallas kernel here")


# speedup vs baseline: 3.4849x; 3.4849x over previous
import functools

import jax
import jax.numpy as jnp
from jax import lax
from jax.experimental import pallas as pl
from jax.experimental.pallas import tpu as pltpu
from jax.experimental.pallas import tpu_sc as plsc

N = 10000
E = 320000
D = 128
OUT = 2
OUTP = 16

NC = 2
NS = 16
NW = NC * NS
NP = 10240
RPS = NP // NS
JUNK = N
C = 64
K = 160
EP = NW * K * C
TBLK = NP // 4


def _sc_agg(with_deg):
    out_type = [jax.ShapeDtypeStruct((NC, NP, D), jnp.float32)]
    scratch = [
        pltpu.VMEM((4, 2, C), jnp.int32),
        pltpu.VMEM((2, C, D), jnp.float32),
        pltpu.VMEM_SHARED((NP, D), jnp.float32),
        pltpu.SemaphoreType.DMA,
        pltpu.SemaphoreType.DMA,
        pltpu.SemaphoreType.DMA,
        pltpu.SemaphoreType.DMA,
        pltpu.SemaphoreType.DMA,
        pltpu.SemaphoreType.DMA,
    ]
    if with_deg:
        out_type.append(jax.ShapeDtypeStruct((NC, NP), jnp.float32))
        scratch += [
            pltpu.VMEM((C,), jnp.float32),
            pltpu.VMEM((RPS,), jnp.float32),
            pltpu.VMEM_SHARED((NP,), jnp.float32),
        ]

    mesh = plsc.VectorSubcoreMesh(core_axis_name="c", subcore_axis_name="s")

    @functools.partial(pl.kernel, out_type=out_type, mesh=mesh,
                       scratch_types=scratch)
    def body(tab_hbm, idx_hbm, *rest):
        if with_deg:
            (agg_hbm, deg_hbm, idx_v, rows_v, acc_sh,
             is0, is1, is2, is3, rs0, rs1, ones_v, zdeg_v, deg_sh) = rest
        else:
            (agg_hbm, idx_v, rows_v, acc_sh,
             is0, is1, is2, is3, rs0, rs1) = rest
        isem = (is0, is1, is2, is3)
        rsem = (rs0, rs1)
        cid = lax.axis_index("c")
        sid = lax.axis_index("s")
        w = sid * NC + cid
        sl = pl.ds(sid * RPS, RPS)

        z16 = jnp.zeros((16,), jnp.float32)

        @pl.loop(0, C)
        def _zrow(r):
            for co in range(D // 16):
                rows_v[0, r, pl.ds(co * 16, 16)] = z16

        for t in range(RPS // C):
            pltpu.sync_copy(rows_v.at[0],
                            acc_sh.at[pl.ds(sid * RPS + t * C, C)])
        if with_deg:
            for co in range(C // 16):
                ones_v[pl.ds(co * 16, 16)] = jnp.ones((16,), jnp.float32)

            @pl.loop(0, RPS // 16)
            def _zdeg(i):
                zdeg_v[pl.ds(i * 16, 16)] = z16

            pltpu.sync_copy(zdeg_v, deg_sh.at[sl])
        plsc.subcore_barrier()

        for s in range(4):
            pltpu.async_copy(idx_hbm.at[w, s], idx_v.at[s], isem[s])
        for b in range(2):
            pltpu.make_async_copy(
                idx_hbm.at[w, b], idx_v.at[b], isem[b]).wait()
            pltpu.async_copy(tab_hbm.at[idx_v.at[b, 0]], rows_v.at[b],
                             rsem[b])

        def step(j):
            for b in range(4):
                jj = j + b
                r = b % 2
                pltpu.make_async_copy(
                    tab_hbm.at[idx_v.at[b, 0]], rows_v.at[r],
                    rsem[r]).wait()
                pltpu.sync_copy(rows_v.at[r], acc_sh.at[idx_v.at[b, 1]],
                                add=True)
                if with_deg:
                    pltpu.sync_copy(ones_v, deg_sh.at[idx_v.at[b, 1]],
                                    add=True)

                @pl.when(jj + 4 < K)
                def _():
                    pltpu.async_copy(idx_hbm.at[w, jj + 4], idx_v.at[b],
                                     isem[b])

                @pl.when(jj + 2 < K)
                def _():
                    s2 = (b + 2) % 4
                    pltpu.make_async_copy(
                        idx_hbm.at[w, jj + 2], idx_v.at[s2],
                        isem[s2]).wait()
                    pltpu.async_copy(tab_hbm.at[idx_v.at[s2, 0]],
                                     rows_v.at[r], rsem[r])

        pl.loop(0, K, step=4)(step)
        plsc.subcore_barrier()

        pltpu.sync_copy(acc_sh.at[sl], agg_hbm.at[cid, sl])
        if with_deg:
            pltpu.sync_copy(deg_sh.at[sl], deg_hbm.at[cid, sl])

    return body


_sc_agg0 = _sc_agg(True)
_sc_agg1 = _sc_agg(False)


def _mm0_body(x_ref, wn_ref, ws_ref, xp_ref, hs_ref):
    x = x_ref[...]
    dims = (((1,), (1,)), ((), ()))
    xp_ref[...] = lax.dot_general(x, wn_ref[...], dims,
                                  preferred_element_type=jnp.float32)
    hs_ref[...] = lax.dot_general(x, ws_ref[...], dims,
                                  preferred_element_type=jnp.float32)


def _mid_body(hs_ref, agg_ref, deg_ref, b0_ref, ws1_ref, b1_ref,
              h_ref, hs1_ref):
    deg = deg_ref[0] + deg_ref[1]
    inv = 1.0 / jnp.maximum(deg, 1.0)
    h = hs_ref[...] + (agg_ref[0] + agg_ref[1]) * inv[:, None] + b0_ref[...]
    h = jnp.maximum(h, 0.0)
    h_ref[...] = h
    dims = (((1,), (1,)), ((), ()))
    hs1_ref[...] = lax.dot_general(h, ws1_ref[...], dims,
                                   preferred_element_type=jnp.float32) + b1_ref[...]


def _fin_body(hs1_ref, agg_ref, deg_ref, wn1_ref, out_ref):
    deg = deg_ref[0] + deg_ref[1]
    inv = 1.0 / jnp.maximum(deg, 1.0)
    hn = (agg_ref[0] + agg_ref[1]) * inv[:, None]
    dims = (((1,), (1,)), ((), ()))
    out_ref[...] = hs1_ref[...] + lax.dot_general(
        hn, wn1_ref[...], dims, preferred_element_type=jnp.float32)


def kernel(x, edge_index, W_self0, W_neigh0, b0, W_self1, W_neigh1, b1):
    f32 = jnp.float32
    src = edge_index[0].astype(jnp.int32)
    dst = edge_index[1].astype(jnp.int32)
    pad = EP - E
    srcs = jnp.concatenate([src, jnp.zeros((pad,), jnp.int32)]).reshape(NW, K, C)
    dsts = jnp.concatenate([dst, jnp.full((pad,), JUNK, jnp.int32)]).reshape(NW, K, C)
    idx = jnp.stack([srcs, dsts], axis=2)
    x_p = jnp.pad(x, ((0, NP - N), (0, 0)))
    wn1p = jnp.pad(W_neigh1, ((0, OUTP - OUT), (0, 0)))
    ws1p = jnp.pad(W_self1, ((0, OUTP - OUT), (0, 0)))
    b1p = jnp.pad(b1, (0, OUTP - OUT)).reshape(1, OUTP)
    b0r = b0.reshape(1, D)

    xp, hs0 = pl.pallas_call(
        _mm0_body,
        grid=(NP // TBLK,),
        in_specs=[
            pl.BlockSpec((TBLK, D), lambda i: (i, 0)),
            pl.BlockSpec((D, D), lambda i: (0, 0)),
            pl.BlockSpec((D, D), lambda i: (0, 0)),
        ],
        out_specs=[
            pl.BlockSpec((TBLK, D), lambda i: (i, 0)),
            pl.BlockSpec((TBLK, D), lambda i: (i, 0)),
        ],
        out_shape=[
            jax.ShapeDtypeStruct((NP, D), f32),
            jax.ShapeDtypeStruct((NP, D), f32),
        ],
    )(x_p, W_neigh0, W_self0)

    aggp0, degp = _sc_agg0(xp, idx)

    h, hs1 = pl.pallas_call(
        _mid_body,
        grid=(NP // TBLK,),
        in_specs=[
            pl.BlockSpec((TBLK, D), lambda i: (i, 0)),
            pl.BlockSpec((NC, TBLK, D), lambda i: (0, i, 0)),
            pl.BlockSpec((NC, TBLK), lambda i: (0, i)),
            pl.BlockSpec((1, D), lambda i: (0, 0)),
            pl.BlockSpec((OUTP, D), lambda i: (0, 0)),
            pl.BlockSpec((1, OUTP), lambda i: (0, 0)),
        ],
        out_specs=[
            pl.BlockSpec((TBLK, D), lambda i: (i, 0)),
            pl.BlockSpec((TBLK, OUTP), lambda i: (i, 0)),
        ],
        out_shape=[
            jax.ShapeDtypeStruct((NP, D), f32),
            jax.ShapeDtypeStruct((NP, OUTP), f32),
        ],
    )(hs0, aggp0, degp, b0r, ws1p, b1p)

    (aggph,) = _sc_agg1(h, idx)

    outp = pl.pallas_call(
        _fin_body,
        grid=(NP // TBLK,),
        in_specs=[
            pl.BlockSpec((TBLK, OUTP), lambda i: (i, 0)),
            pl.BlockSpec((NC, TBLK, D), lambda i: (0, i, 0)),
            pl.BlockSpec((NC, TBLK), lambda i: (0, i)),
            pl.BlockSpec((OUTP, D), lambda i: (0, 0)),
        ],
        out_specs=pl.BlockSpec((TBLK, OUTP), lambda i: (i, 0)),
        out_shape=jax.ShapeDtypeStruct((NP, OUTP), f32),
    )(hs1, aggph, degp, wn1p)

    return outp[:N, :OUT]
